# Initial kernel scaffold; baseline (speedup 1.0000x reference)
import jax
import jax.numpy as jnp
from jax.experimental import pallas as pl


def kernel(hash_indices, hidden_states, table, w_v, w_k, g_k, g_h, norms_w, conv_w):
    raise NotImplementedError("write your pallas kernel here")


# speedup vs baseline: 6.0962x; 6.0962x over previous
import functools

import numpy as np
import jax
import jax.numpy as jnp
from jax import lax
from jax.experimental import pallas as pl
from jax.experimental.pallas import tpu as pltpu
from jax.experimental.pallas import tpu_sc as plsc

_PRIMES = [49999, 49993, 49991, 49957, 49943, 49939, 49937, 49927]
_H = len(_PRIMES)
_HC = 4
_HID = 1024
_DH = 128
_KSZ = 4
_DIL = 3
_EPS_G = float(np.finfo(np.float32).eps)
_EPS_C = 1e-5
_OFFSETS = np.concatenate(
    [[0], np.cumsum(np.asarray(_PRIMES[:-1], dtype=np.int64))]
).astype(np.int32)

_NW = 32
_CHUNK = 128


def _sc_gather(table, idx):
    nw, nchunks, c = idx.shape
    rows_total = nw * nchunks * c
    per_worker = nchunks * c
    mesh = plsc.VectorSubcoreMesh(core_axis_name="c", subcore_axis_name="s")

    @functools.partial(
        pl.kernel,
        out_type=jax.ShapeDtypeStruct((rows_total, _DH), jnp.float32),
        mesh=mesh,
        scratch_types=[
            pltpu.VMEM((nchunks, c), jnp.int32),
            pltpu.VMEM((c, _DH), jnp.float32),
            pltpu.VMEM((c, _DH), jnp.float32),
            pltpu.SemaphoreType.DMA,
            pltpu.SemaphoreType.DMA,
        ],
    )
    def gather_kernel(table_hbm, idx_hbm, out_hbm, idx_v, buf0, buf1, sem0, sem1):
        wid = lax.axis_index("s") * 2 + lax.axis_index("c")
        base = wid * per_worker
        pltpu.sync_copy(idx_hbm.at[wid], idx_v)

        bufs = (buf0, buf1)
        sems = (sem0, sem1)
        pltpu.async_copy(table_hbm.at[idx_v.at[0]], buf0, sem0)

        def step(j, _):
            def even_odd(parity):
                buf, sem = bufs[parity], sems[parity]
                nbuf, nsem = bufs[1 - parity], sems[1 - parity]

                @pl.when(j + 1 < nchunks)
                def _():
                    pltpu.async_copy(table_hbm.at[idx_v.at[j + 1]], nbuf, nsem)

                pltpu.make_async_copy(table_hbm.at[idx_v.at[j]], buf, sem).wait()
                pltpu.sync_copy(buf, out_hbm.at[pl.ds(base + j * c, c)])

            @pl.when(j % 2 == 0)
            def _():
                even_odd(0)

            @pl.when(j % 2 == 1)
            def _():
                even_odd(1)

            return 0

        lax.fori_loop(0, nchunks, step, 0)

    return gather_kernel(table, idx)


def _dense(emb, hid, wcat, g_k, g_h, norms_w, convw):
    n = emb.shape[0]
    t = 256
    grid = n // t
    cpb = 2048 // t
    halo = 16

    def body(emb_ref, hid_ref, w_ref, gk_ref, gh_ref, nw_ref, cw_ref,
             out_ref, tail_ref):
        i = pl.program_id(0)
        seq_start = (i % cpb) == 0
        p = jnp.dot(emb_ref[...], w_ref[...],
                    preferred_element_type=jnp.float32)
        value = p[:, :_HID]
        for m in range(_HC):
            k = p[:, _HID * (m + 1):_HID * (m + 2)]
            nk = k * lax.rsqrt(jnp.mean(k * k, axis=-1, keepdims=True)
                               + _EPS_G) * gk_ref[m][None, :]
            q = hid_ref[:, _HID * m:_HID * (m + 1)]
            nq = q * lax.rsqrt(jnp.mean(q * q, axis=-1, keepdims=True)
                               + _EPS_G) * gh_ref[m][None, :]
            g = jnp.sum(nk * nq, axis=-1, keepdims=True) / np.sqrt(float(_HID))
            g = jnp.sqrt(jnp.clip(jnp.abs(g), 1e-6, None)) * jnp.sign(g)
            gate = jax.nn.sigmoid(g)
            gated = gate * value
            xs = gated * lax.rsqrt(jnp.mean(gated * gated, axis=-1,
                                            keepdims=True) + _EPS_C) \
                * nw_ref[m][None, :]
            prev_tail = jnp.where(seq_start, 0.0, tail_ref[m])
            xfull = jnp.concatenate([prev_tail, xs], axis=0)
            co = jnp.zeros_like(xs)
            for kk in range(_KSZ):
                off = halo - (_KSZ - 1) * _DIL + _DIL * kk
                co = co + xfull[off:off + t, :] \
                    * cw_ref[kk, _HID * m:_HID * (m + 1)][None, :]
            co = co * jax.nn.sigmoid(co)
            tail_ref[m] = xs[t - halo:, :]
            out_ref[:, _HID * m:_HID * (m + 1)] = q + co + gated

    return pl.pallas_call(
        body,
        grid=(grid,),
        in_specs=[
            pl.BlockSpec((t, _H * _DH), lambda i: (i, 0)),
            pl.BlockSpec((t, _HC * _HID), lambda i: (i, 0)),
            pl.BlockSpec(((_H * _DH), (1 + _HC) * _HID), lambda i: (0, 0)),
            pl.BlockSpec((_HC, _HID), lambda i: (0, 0)),
            pl.BlockSpec((_HC, _HID), lambda i: (0, 0)),
            pl.BlockSpec((_HC, _HID), lambda i: (0, 0)),
            pl.BlockSpec((_KSZ, _HC * _HID), lambda i: (0, 0)),
        ],
        out_specs=pl.BlockSpec((t, _HC * _HID), lambda i: (i, 0)),
        out_shape=jax.ShapeDtypeStruct((n, _HC * _HID), jnp.float32),
        scratch_shapes=[pltpu.VMEM((_HC, 16, _HID), jnp.float32)],
    )(emb, hid, wcat, g_k, g_h, norms_w, convw)


def kernel(hash_indices, hidden_states, table, w_v, w_k, g_k, g_h,
           norms_w, conv_w):
    b, s, h = hash_indices.shape
    n = b * s
    idx = hash_indices.astype(jnp.int32) + jnp.asarray(_OFFSETS, jnp.int32)
    idx = idx.reshape(_NW, -1, _CHUNK)
    emb = _sc_gather(table, idx).reshape(n, _H * _DH)
    hid = hidden_states.reshape(n, _HC * _HID)
    wcat = jnp.concatenate(
        [w_v, w_k.reshape(_HC * _HID, _H * _DH)], axis=0).T
    convw = conv_w.reshape(_HC * _HID, _KSZ).T
    out = _dense(emb, hid, wcat, g_k, g_h, norms_w, convw)
    return out.reshape(b, s, _HC, _HID)


# speedup vs baseline: 6.2473x; 1.0248x over previous
import functools

import numpy as np
import jax
import jax.numpy as jnp
from jax import lax
from jax.experimental import pallas as pl
from jax.experimental.pallas import tpu as pltpu
from jax.experimental.pallas import tpu_sc as plsc

_PRIMES = [49999, 49993, 49991, 49957, 49943, 49939, 49937, 49927]
_H = len(_PRIMES)
_HC = 4
_HID = 1024
_DH = 128
_KSZ = 4
_DIL = 3
_EPS_G = float(np.finfo(np.float32).eps)
_EPS_C = 1e-5
_OFFSETS = np.concatenate(
    [[0], np.cumsum(np.asarray(_PRIMES[:-1], dtype=np.int64))]
).astype(np.int32)

_NW = 32
_CHUNK = 128


def _sc_gather(table, idx):
    nw, nchunks, c = idx.shape
    rows_total = nw * nchunks * c
    per_worker = nchunks * c
    mesh = plsc.VectorSubcoreMesh(core_axis_name="c", subcore_axis_name="s")

    @functools.partial(
        pl.kernel,
        out_type=jax.ShapeDtypeStruct((rows_total, _DH), jnp.float32),
        mesh=mesh,
        scratch_types=[
            pltpu.VMEM((nchunks, c), jnp.int32),
            pltpu.VMEM((c, _DH), jnp.float32),
            pltpu.VMEM((c, _DH), jnp.float32),
            pltpu.SemaphoreType.DMA,
            pltpu.SemaphoreType.DMA,
        ],
    )
    def gather_kernel(table_hbm, idx_hbm, out_hbm, idx_v, buf0, buf1, sem0, sem1):
        wid = lax.axis_index("s") * 2 + lax.axis_index("c")
        base = wid * per_worker
        pltpu.sync_copy(idx_hbm.at[wid], idx_v)

        bufs = (buf0, buf1)
        sems = (sem0, sem1)
        pltpu.async_copy(table_hbm.at[idx_v.at[0]], buf0, sem0)

        def step(j, _):
            def even_odd(parity):
                buf, sem = bufs[parity], sems[parity]
                nbuf, nsem = bufs[1 - parity], sems[1 - parity]

                @pl.when(j + 1 < nchunks)
                def _():
                    pltpu.async_copy(table_hbm.at[idx_v.at[j + 1]], nbuf, nsem)

                pltpu.make_async_copy(table_hbm.at[idx_v.at[j]], buf, sem).wait()
                pltpu.sync_copy(buf, out_hbm.at[pl.ds(base + j * c, c)])

            @pl.when(j % 2 == 0)
            def _():
                even_odd(0)

            @pl.when(j % 2 == 1)
            def _():
                even_odd(1)

            return 0

        lax.fori_loop(0, nchunks, step, 0)

    return gather_kernel(table, idx)


def _dense(emb, hid, wcat, g_k, g_h, norms_w, convw):
    n = emb.shape[0]
    t = 256
    grid = n // t
    cpb = 2048 // t
    halo = 16

    def body(emb_ref, hid_ref, w_ref, gk_ref, gh_ref, nw_ref, cw_ref,
             out_ref, tail_ref):
        i = pl.program_id(0)
        seq_start = (i % cpb) == 0
        eb = emb_ref[...].astype(jnp.bfloat16)
        p = lax.dot_general(eb, w_ref[...],
                            (((1,), (1,)), ((), ())),
                            preferred_element_type=jnp.float32)
        value = p[:, :_HID]
        for m in range(_HC):
            k = p[:, _HID * (m + 1):_HID * (m + 2)]
            nk = k * lax.rsqrt(jnp.mean(k * k, axis=-1, keepdims=True)
                               + _EPS_G) * gk_ref[m][None, :]
            q = hid_ref[:, _HID * m:_HID * (m + 1)]
            nq = q * lax.rsqrt(jnp.mean(q * q, axis=-1, keepdims=True)
                               + _EPS_G) * gh_ref[m][None, :]
            g = jnp.sum(nk * nq, axis=-1, keepdims=True) / np.sqrt(float(_HID))
            g = jnp.sqrt(jnp.clip(jnp.abs(g), 1e-6, None)) * jnp.sign(g)
            gate = jax.nn.sigmoid(g)
            gated = gate * value
            xs = gated * lax.rsqrt(jnp.mean(gated * gated, axis=-1,
                                            keepdims=True) + _EPS_C) \
                * nw_ref[m][None, :]
            prev_tail = jnp.where(seq_start, 0.0, tail_ref[m])
            xfull = jnp.concatenate([prev_tail, xs], axis=0)
            co = jnp.zeros_like(xs)
            for kk in range(_KSZ):
                off = halo - (_KSZ - 1) * _DIL + _DIL * kk
                co = co + xfull[off:off + t, :] \
                    * cw_ref[kk, _HID * m:_HID * (m + 1)][None, :]
            co = co * jax.nn.sigmoid(co)
            tail_ref[m] = xs[t - halo:, :]
            out_ref[:, _HID * m:_HID * (m + 1)] = q + co + gated

    return pl.pallas_call(
        body,
        grid=(grid,),
        in_specs=[
            pl.BlockSpec((t, _H * _DH), lambda i: (i, 0)),
            pl.BlockSpec((t, _HC * _HID), lambda i: (i, 0)),
            pl.BlockSpec(((1 + _HC) * _HID, _H * _DH), lambda i: (0, 0)),
            pl.BlockSpec((_HC, _HID), lambda i: (0, 0)),
            pl.BlockSpec((_HC, _HID), lambda i: (0, 0)),
            pl.BlockSpec((_HC, _HID), lambda i: (0, 0)),
            pl.BlockSpec((_KSZ, _HC * _HID), lambda i: (0, 0)),
        ],
        out_specs=pl.BlockSpec((t, _HC * _HID), lambda i: (i, 0)),
        out_shape=jax.ShapeDtypeStruct((n, _HC * _HID), jnp.float32),
        scratch_shapes=[pltpu.VMEM((_HC, 16, _HID), jnp.float32)],
    )(emb, hid, wcat, g_k, g_h, norms_w, convw)


def kernel(hash_indices, hidden_states, table, w_v, w_k, g_k, g_h,
           norms_w, conv_w):
    b, s, h = hash_indices.shape
    n = b * s
    idx = hash_indices.astype(jnp.int32) + jnp.asarray(_OFFSETS, jnp.int32)
    idx = idx.reshape(_NW, -1, _CHUNK)
    emb = _sc_gather(table, idx).reshape(n, _H * _DH)
    hid = hidden_states.reshape(n, _HC * _HID)
    wcat = jnp.concatenate(
        [w_v, w_k.reshape(_HC * _HID, _H * _DH)], axis=0
    ).astype(jnp.bfloat16)
    convw = conv_w.reshape(_HC * _HID, _KSZ).T
    out = _dense(emb, hid, wcat, g_k, g_h, norms_w, convw)
    return out.reshape(b, s, _HC, _HID)
